# Initial kernel scaffold; baseline (speedup 1.0000x reference)
import jax
import jax.numpy as jnp
from jax.experimental import pallas as pl


def kernel(emb_0, emb_1, emb_2, emb_3, W_0, W_1, W_2, W_3, b_0, b_1, b_2, b_3, idx):
    raise NotImplementedError("write your pallas kernel here")


# speedup vs baseline: 5.6369x; 5.6369x over previous
import functools

import jax
import jax.numpy as jnp
from jax.experimental import pallas as pl

_K = 20
_ALPHA = 3.0
_ONE_BITS = 0x3F800000


def _nv_kernel(e1_ref, e2_ref, w_ref, b_ref, p_ref, q_ref):
    w = w_ref[0]
    b = b_ref[0]
    e1 = e1_ref[0]
    e2 = e2_ref[0]
    dn = (((1,), (1,)), ((), ()))
    p_ref[0] = jnp.tanh(
        _ALPHA * (jax.lax.dot_general(e1, w, dn, preferred_element_type=jnp.float32) + b))
    q_ref[0] = jnp.tanh(
        _ALPHA * (jax.lax.dot_general(e2, w, dn, preferred_element_type=jnp.float32) + b))


def _adj_topk_kernel(p0_ref, p1_ref, q0_ref, q1_ref, out_ref, *, n_cols):
    dn = (((1,), (1,)), ((), ()))
    s0 = jax.lax.dot_general(p0_ref[0], q0_ref[0], dn, preferred_element_type=jnp.float32)
    s1 = jax.lax.dot_general(p1_ref[0], q1_ref[0], dn, preferred_element_type=jnp.float32)
    v = jnp.concatenate([s0, s1], axis=1)
    v = jnp.maximum(jnp.tanh(_ALPHA * v), 0.0)
    u = jax.lax.bitcast_convert_type(v, jnp.int32)
    rows = v.shape[0]

    lo0 = jnp.zeros((rows, 1), jnp.int32)
    hi0 = jnp.full((rows, 1), _ONE_BITS + 1, jnp.int32)

    def body(_, carry):
        lo, hi = carry
        mid = (lo + hi) // 2
        cnt = jnp.sum((u >= mid).astype(jnp.int32), axis=1, keepdims=True)
        ge = cnt >= _K
        return jnp.where(ge, mid, lo), jnp.where(ge, hi, mid)

    tau_u, _ = jax.lax.fori_loop(0, 31, body, (lo0, hi0))

    gt = u > tau_u
    c_gt = jnp.sum(gt.astype(jnp.int32), axis=1, keepdims=True)
    tie = (u == tau_u).astype(jnp.int32)
    inc = tie
    shift = 1
    while shift < n_cols:
        shifted = jnp.concatenate(
            [jnp.zeros((rows, shift), jnp.int32), inc[:, :-shift]], axis=1)
        inc = inc + shifted
        shift *= 2
    ex_rank = inc - tie
    sel = gt | ((tie == 1) & (ex_rank < (_K - c_gt)))
    out_ref[...] = jnp.where(sel, v, 0.0)


def kernel(emb_0, emb_1, emb_2, emb_3, W_0, W_1, W_2, W_3, b_0, b_1, b_2, b_3, idx):
    L, D = emb_0.shape
    n = 2 * L
    embs = jnp.stack([emb_0, emb_1, emb_2, emb_3])
    Ws = jnp.stack([W_0, W_1, W_2, W_3])
    bs = jnp.stack([b_0, b_1, b_2, b_3]).reshape(4, 1, D)

    p_tables, q_tables = pl.pallas_call(
        _nv_kernel,
        grid=(4,),
        in_specs=[
            pl.BlockSpec((1, L, D), lambda m: (m, 0, 0)),
            pl.BlockSpec((1, L, D), lambda m: ((m % 2) * 2 + m // 2, 0, 0)),
            pl.BlockSpec((1, D, D), lambda m: (m, 0, 0)),
            pl.BlockSpec((1, 1, D), lambda m: (m, 0, 0)),
        ],
        out_specs=[
            pl.BlockSpec((1, L, D), lambda m: (m, 0, 0)),
            pl.BlockSpec((1, L, D), lambda m: (m, 0, 0)),
        ],
        out_shape=[
            jax.ShapeDtypeStruct((4, L, D), jnp.float32),
            jax.ShapeDtypeStruct((4, L, D), jnp.float32),
        ],
    )(embs, embs, Ws, bs)

    rows_per_block = 200 if L % 200 == 0 else L
    blocks_per_half = L // rows_per_block

    body = functools.partial(_adj_topk_kernel, n_cols=n)
    out = pl.pallas_call(
        body,
        grid=(2 * blocks_per_half,),
        in_specs=[
            pl.BlockSpec((1, rows_per_block, D),
                         lambda b: (2 * (b // blocks_per_half), b % blocks_per_half, 0)),
            pl.BlockSpec((1, rows_per_block, D),
                         lambda b: (2 * (b // blocks_per_half) + 1, b % blocks_per_half, 0)),
            pl.BlockSpec((1, L, D), lambda b: (2 * (b // blocks_per_half), 0, 0)),
            pl.BlockSpec((1, L, D), lambda b: (2 * (b // blocks_per_half) + 1, 0, 0)),
        ],
        out_specs=pl.BlockSpec((rows_per_block, n), lambda b: (b, 0)),
        out_shape=jax.ShapeDtypeStruct((n, n), jnp.float32),
    )(p_tables, p_tables, q_tables, q_tables)
    return out


# speedup vs baseline: 56.0320x; 9.9403x over previous
import functools

import jax
import jax.numpy as jnp
from jax.experimental import pallas as pl
from jax.experimental.pallas import tpu as pltpu

_K = 20
_ALPHA = 3.0
_ONE_BITS = 0x3F800000


def _nv_kernel(e1_ref, e2_ref, w_ref, b_ref, p_ref, q_ref):
    w = w_ref[0]
    b = b_ref[0]
    e1 = e1_ref[0]
    e2 = e2_ref[0]
    dn = (((1,), (1,)), ((), ()))
    p_ref[0] = jnp.tanh(
        _ALPHA * (jax.lax.dot_general(e1, w, dn, preferred_element_type=jnp.float32) + b))
    q_ref[0] = jnp.tanh(
        _ALPHA * (jax.lax.dot_general(e2, w, dn, preferred_element_type=jnp.float32) + b))


def _adj_topk_kernel(p0_ref, p1_ref, q0_ref, q1_ref, out_ref, tau_ref, cgt_ref,
                     *, n_cols):
    dn = (((1,), (1,)), ((), ()))
    s0 = jax.lax.dot_general(p0_ref[0], q0_ref[0], dn, preferred_element_type=jnp.float32)
    s1 = jax.lax.dot_general(p1_ref[0], q1_ref[0], dn, preferred_element_type=jnp.float32)
    v = jnp.concatenate([s0, s1], axis=1)
    v = jnp.maximum(jnp.tanh(_ALPHA * v), 0.0)
    u = jax.lax.bitcast_convert_type(v, jnp.int32)
    rows = v.shape[0]

    ones_cnt = jnp.sum((u == _ONE_BITS).astype(jnp.int32), axis=1, keepdims=True)
    p_fast = jnp.min(ones_cnt) >= _K

    @pl.when(p_fast)
    def _():
        tau_ref[...] = jnp.full((rows, 1), _ONE_BITS, jnp.int32)
        cgt_ref[...] = jnp.zeros((rows, 1), jnp.int32)

    @pl.when(jnp.logical_not(p_fast))
    def _():
        lo0 = jnp.zeros((rows, 1), jnp.int32)
        hi0 = jnp.full((rows, 1), _ONE_BITS + 1, jnp.int32)

        def body(_, carry):
            lo, hi = carry
            mid = (lo + hi) // 2
            cnt = jnp.sum((u >= mid).astype(jnp.int32), axis=1, keepdims=True)
            ge = cnt >= _K
            return jnp.where(ge, mid, lo), jnp.where(ge, hi, mid)

        tau_u, _unused = jax.lax.fori_loop(0, 31, body, (lo0, hi0))
        tau_ref[...] = tau_u
        cgt_ref[...] = jnp.sum((u > tau_u).astype(jnp.int32), axis=1, keepdims=True)

    tau_u = tau_ref[...]
    need = _K - cgt_ref[...]
    tie = u == tau_u
    gt = u > tau_u

    def _write_with_window(w):
        def f():
            t = tie[:, :w]
            inc = t.astype(jnp.int32)
            shift = 1
            while shift < w:
                inc = inc + jnp.concatenate(
                    [jnp.zeros((rows, shift), jnp.int32), inc[:, :-shift]], axis=1)
                shift *= 2
            selw = t & ((inc - t.astype(jnp.int32)) < need)
            if w == n_cols:
                sel = gt | selw
            else:
                sel = gt | jnp.concatenate(
                    [selw, jnp.zeros((rows, n_cols - w), jnp.bool_)], axis=1)
            out_ref[...] = jnp.where(sel, v, 0.0)
        return f

    windows = [w for w in (128, 1024) if w < n_cols] + [n_cols]
    covered = [
        jnp.all(jnp.sum(tie[:, :w].astype(jnp.int32), axis=1, keepdims=True) >= need)
        for w in windows[:-1]
    ]
    prev_ok = None
    for widx, w in enumerate(windows):
        if widx == 0:
            pred = covered[0] if len(windows) > 1 else jnp.bool_(True)
            prev_ok = pred
        elif widx < len(windows) - 1:
            pred = jnp.logical_and(jnp.logical_not(prev_ok), covered[widx])
            prev_ok = jnp.logical_or(prev_ok, covered[widx])
        else:
            pred = jnp.logical_not(prev_ok)
        pl.when(pred)(_write_with_window(w))


def kernel(emb_0, emb_1, emb_2, emb_3, W_0, W_1, W_2, W_3, b_0, b_1, b_2, b_3, idx):
    L, D = emb_0.shape
    n = 2 * L
    embs = jnp.stack([emb_0, emb_1, emb_2, emb_3])
    Ws = jnp.stack([W_0, W_1, W_2, W_3])
    bs = jnp.stack([b_0, b_1, b_2, b_3]).reshape(4, 1, D)

    p_tables, q_tables = pl.pallas_call(
        _nv_kernel,
        grid=(4,),
        in_specs=[
            pl.BlockSpec((1, L, D), lambda m: (m, 0, 0)),
            pl.BlockSpec((1, L, D), lambda m: ((m % 2) * 2 + m // 2, 0, 0)),
            pl.BlockSpec((1, D, D), lambda m: (m, 0, 0)),
            pl.BlockSpec((1, 1, D), lambda m: (m, 0, 0)),
        ],
        out_specs=[
            pl.BlockSpec((1, L, D), lambda m: (m, 0, 0)),
            pl.BlockSpec((1, L, D), lambda m: (m, 0, 0)),
        ],
        out_shape=[
            jax.ShapeDtypeStruct((4, L, D), jnp.float32),
            jax.ShapeDtypeStruct((4, L, D), jnp.float32),
        ],
    )(embs, embs, Ws, bs)

    rows_per_block = 200 if L % 200 == 0 else L
    blocks_per_half = L // rows_per_block

    body = functools.partial(_adj_topk_kernel, n_cols=n)
    out = pl.pallas_call(
        body,
        grid=(2 * blocks_per_half,),
        in_specs=[
            pl.BlockSpec((1, rows_per_block, D),
                         lambda b: (2 * (b // blocks_per_half), b % blocks_per_half, 0)),
            pl.BlockSpec((1, rows_per_block, D),
                         lambda b: (2 * (b // blocks_per_half) + 1, b % blocks_per_half, 0)),
            pl.BlockSpec((1, L, D), lambda b: (2 * (b // blocks_per_half), 0, 0)),
            pl.BlockSpec((1, L, D), lambda b: (2 * (b // blocks_per_half) + 1, 0, 0)),
        ],
        out_specs=pl.BlockSpec((rows_per_block, n), lambda b: (b, 0)),
        out_shape=jax.ShapeDtypeStruct((n, n), jnp.float32),
        scratch_shapes=[
            pltpu.VMEM((rows_per_block, 1), jnp.int32),
            pltpu.VMEM((rows_per_block, 1), jnp.int32),
        ],
    )(p_tables, p_tables, q_tables, q_tables)
    return out
